# baseline (speedup 1.0000x reference)
import jax
import jax.numpy as jnp
from jax.experimental import pallas as pl
from jax.experimental.pallas import tpu as pltpu

B, S, D = 4, 8192, 1024
HID = 512
NUM_OUT = 8

CHUNKS = [(0, 0, 512, 0), (0, 512, 512, 1), (0, 1024, 1024, 2),
          (0, 2048, 2048, 3), (0, 4096, 4096, 4)]
for _b in range(1, B):
    CHUNKS.append((_b, 0, 4096, 3))
    CHUNKS.append((_b, 4096, 4096, 4))
NBUF = 5
_BUF_ROWS = (512, 512, 1024, 4096, 4096)


def _router_kernel(x_ref, wp_ref, w1_ref, b1_ref, w2_ref, b2_ref, out_ref,
                   b0, b1_buf, b2_buf, b3, b4, sems):
    bufs = (b0, b1_buf, b2_buf, b3, b4)

    def dma(ci):
        bt, row0, n, bi = CHUNKS[ci]
        return pltpu.make_async_copy(
            x_ref.at[bt, pl.ds(row0, n), :],
            bufs[bi].at[pl.ds(0, n), :], sems.at[bi])

    next_use = {}
    for ci in range(len(CHUNKS) - 1, -1, -1):
        next_use[ci] = None
        for cj in range(ci + 1, len(CHUNKS)):
            if CHUNKS[cj][3] == CHUNKS[ci][3]:
                next_use[ci] = cj
                break

    started = set()
    for ci, ch in enumerate(CHUNKS):
        if ch[3] not in {CHUNKS[cj][3] for cj in range(ci)}:
            dma(ci).start()
            started.add(ci)

    wp_row = wp_ref[...].reshape(1, D)
    parts = [[] for _ in range(B)]
    for ci, (bt, row0, n, bi) in enumerate(CHUNKS):
        dma(ci).wait()
        xb = bufs[bi][pl.ds(0, n), :]
        s = jax.lax.dot_general(
            wp_row, xb, (((1,), (1,)), ((), ())),
            preferred_element_type=jnp.float32)
        m_g = jnp.max(s)
        p = jnp.exp(s - m_g)
        l_g = jnp.sum(p)
        acc_g = jnp.dot(p, xb, preferred_element_type=jnp.float32)
        parts[bt].append((m_g, l_g, acc_g))
        nu = next_use[ci]
        if nu is not None and nu not in started:
            dma(nu).start()
            started.add(nu)

    pooled_rows = []
    for bt in range(B):
        ms = [m for m, _, _ in parts[bt]]
        mb = ms[0]
        for mg in ms[1:]:
            mb = jnp.maximum(mb, mg)
        scale = [jnp.exp(mg - mb) for mg in ms]
        lb = sum(sc * lg for sc, (_, lg, _) in zip(scale, parts[bt]))
        accb = sum(sc * ag for sc, (_, _, ag) in zip(scale, parts[bt]))
        pooled_rows.append(accb / lb)

    pooled = jnp.concatenate(pooled_rows, axis=0)
    h = jnp.dot(pooled, w1_ref[...], preferred_element_type=jnp.float32)
    h = jnp.maximum(h + b1_ref[...], 0.0)
    logits = jnp.dot(h, w2_ref[...], preferred_element_type=jnp.float32)
    logits = logits + b2_ref[...]

    col = jax.lax.broadcasted_iota(jnp.int32, (B, NUM_OUT), 1)
    m1 = jnp.max(logits, axis=1, keepdims=True)
    i1 = jnp.min(jnp.where(logits == m1, col, NUM_OUT), axis=1, keepdims=True)
    l2 = jnp.where(col == i1, -jnp.inf, logits)
    m2 = jnp.max(l2, axis=1, keepdims=True)
    i2 = jnp.min(jnp.where(l2 == m2, col, NUM_OUT), axis=1, keepdims=True)
    keep = (col == i1) | (col == i2)
    e = jnp.where(keep, jnp.exp(logits - m1), 0.0)
    out_ref[...] = e / jnp.sum(e, axis=1, keepdims=True)


@jax.jit
def kernel(x, w_pool, b_pool, w1, b1, w2, b2):
    del b_pool
    return pl.pallas_call(
        _router_kernel,
        in_specs=[
            pl.BlockSpec(memory_space=pl.ANY),
            pl.BlockSpec((D, 1), lambda: (0, 0)),
            pl.BlockSpec((D, HID), lambda: (0, 0)),
            pl.BlockSpec((1, HID), lambda: (0, 0)),
            pl.BlockSpec((HID, NUM_OUT), lambda: (0, 0)),
            pl.BlockSpec((1, NUM_OUT), lambda: (0, 0)),
        ],
        out_specs=pl.BlockSpec((B, NUM_OUT), lambda: (0, 0)),
        out_shape=jax.ShapeDtypeStruct((B, NUM_OUT), jnp.float32),
        scratch_shapes=(
            [pltpu.VMEM((n, D), jnp.float32) for n in _BUF_ROWS]
            + [pltpu.SemaphoreType.DMA((NBUF,))]),
    )(x, w_pool, w1, b1.reshape(1, HID), w2, b2.reshape(1, NUM_OUT))


# speedup vs baseline: 1.0439x; 1.0439x over previous
import functools

import jax
import jax.numpy as jnp
from jax.experimental import pallas as pl
from jax.experimental.pallas import tpu as pltpu

B, S, D = 4, 8192, 1024
HID = 512
NUM_OUT = 8
CS = 4096
NC = S // CS


def _router_kernel(x_ref, w_pool_ref, w1_ref, b1_ref, w2_ref, b2_ref,
                   out_ref, pooled_ref, m_ref, l_ref):
    b = pl.program_id(0)
    c = pl.program_id(1)

    @pl.when(c == 0)
    def _init():
        m_ref[0] = -jnp.inf
        l_ref[0] = 0.0

    x_blk = x_ref[0]
    wp_row = w_pool_ref[...].reshape(1, D)
    s = jnp.sum(x_blk * wp_row, axis=1, keepdims=True)
    m_c = jnp.max(s)
    m_prev = m_ref[0]
    m_new = jnp.maximum(m_prev, m_c)
    m_ref[0] = m_new
    alpha = jnp.exp(m_prev - m_new)
    p = jnp.exp(s - m_new)
    l_ref[0] = l_ref[0] * alpha + jnp.sum(p)
    acc_c = jnp.dot(p.T, x_blk, preferred_element_type=jnp.float32)

    @pl.when(c == 0)
    def _first():
        pooled_ref[pl.ds(b, 1), :] = acc_c

    @pl.when(c > 0)
    def _rest():
        pooled_ref[pl.ds(b, 1), :] = pooled_ref[pl.ds(b, 1), :] * alpha + acc_c

    @pl.when(c == NC - 1)
    def _finish_batch():
        pooled_ref[pl.ds(b, 1), :] = pooled_ref[pl.ds(b, 1), :] / l_ref[0]

    @pl.when((b == B - 1) & (c == NC - 1))
    def _mlp():
        pooled = pooled_ref[...]
        h = jnp.dot(pooled, w1_ref[...], preferred_element_type=jnp.float32)
        h = jnp.maximum(h + b1_ref[...], 0.0)
        logits = jnp.dot(h, w2_ref[...], preferred_element_type=jnp.float32)
        logits = logits + b2_ref[...]

        col = jax.lax.broadcasted_iota(jnp.int32, (B, NUM_OUT), 1)
        m1 = jnp.max(logits, axis=1, keepdims=True)
        i1 = jnp.min(jnp.where(logits == m1, col, NUM_OUT), axis=1, keepdims=True)
        l2 = jnp.where(col == i1, -jnp.inf, logits)
        m2 = jnp.max(l2, axis=1, keepdims=True)
        i2 = jnp.min(jnp.where(l2 == m2, col, NUM_OUT), axis=1, keepdims=True)
        keep = (col == i1) | (col == i2)
        e = jnp.where(keep, jnp.exp(logits - m1), 0.0)
        out_ref[...] = e / jnp.sum(e, axis=1, keepdims=True)


@functools.partial(jax.jit, static_argnames=())
def kernel(x, w_pool, b_pool, w1, b1, w2, b2):
    del b_pool
    b1_2d = b1.reshape(1, HID)
    b2_2d = b2.reshape(1, NUM_OUT)
    return pl.pallas_call(
        _router_kernel,
        grid=(B, NC),
        in_specs=[
            pl.BlockSpec((1, CS, D), lambda b, c: (b, c, 0)),
            pl.BlockSpec((D, 1), lambda b, c: (0, 0)),
            pl.BlockSpec((D, HID), lambda b, c: (0, 0)),
            pl.BlockSpec((1, HID), lambda b, c: (0, 0)),
            pl.BlockSpec((HID, NUM_OUT), lambda b, c: (0, 0)),
            pl.BlockSpec((1, NUM_OUT), lambda b, c: (0, 0)),
        ],
        out_specs=pl.BlockSpec((B, NUM_OUT), lambda b, c: (0, 0)),
        out_shape=jax.ShapeDtypeStruct((B, NUM_OUT), jnp.float32),
        scratch_shapes=[
            pltpu.VMEM((B, D), jnp.float32),
            pltpu.SMEM((1,), jnp.float32),
            pltpu.SMEM((1,), jnp.float32),
        ],
    )(x, w_pool, w1, b1_2d, w2, b2_2d)
